# Initial kernel scaffold; baseline (speedup 1.0000x reference)
import jax
import jax.numpy as jnp
from jax.experimental import pallas as pl


def kernel(x, Wq, Wk, edge_index):
    raise NotImplementedError("write your pallas kernel here")


# speedup vs baseline: 2.4515x; 2.4515x over previous
import dataclasses
import functools
import math

import jax
import jax.numpy as jnp
from jax import lax
from jax.experimental import pallas as pl
from jax.experimental.pallas import tpu as pltpu
from jax.experimental.pallas import tpu_sc as plsc

N = 10000
E = 160000
F = 256
HF = 128
SCALE = math.sqrt(F)

NPAD = 10240
EPAD = 163840
CH = 128
NC = 2
NS = 16
P1_CHUNKS = EPAD // (NC * NS) // CH
P2_CHUNKS = EPAD // NS // CH
ROWS_PER_TILE = NPAD // NS

_mesh = plsc.VectorSubcoreMesh(core_axis_name="c", subcore_axis_name="s")

_sc_params = pltpu.CompilerParams()
if "needs_layout_passes" in pltpu.CompilerParams.__dataclass_fields__:
    _sc_params = dataclasses.replace(_sc_params, needs_layout_passes=False)


def _proj_body(x_ref, wq_ref, wk_ref, q0_ref, q1_ref, k0_ref, k1_ref):
    xb = x_ref[...]
    dn = (((1,), (1,)), ((), ()))
    q = lax.dot_general(xb, wq_ref[...], dn,
                        preferred_element_type=jnp.float32,
                        precision=lax.Precision.HIGHEST)
    k = lax.dot_general(xb, wk_ref[...], dn,
                        preferred_element_type=jnp.float32,
                        precision=lax.Precision.HIGHEST)
    q0_ref[...] = q[:, :HF]
    q1_ref[...] = q[:, HF:]
    k0_ref[...] = k[:, :HF]
    k1_ref[...] = k[:, HF:]


def _proj(x_pad, Wq, Wk):
    blk = 1024
    grid = (NPAD // blk,)
    half = jax.ShapeDtypeStruct((NPAD, HF), jnp.float32)
    return pl.pallas_call(
        _proj_body,
        grid=grid,
        in_specs=[
            pl.BlockSpec((blk, F), lambda i: (i, 0)),
            pl.BlockSpec((F, F), lambda i: (0, 0)),
            pl.BlockSpec((F, F), lambda i: (0, 0)),
        ],
        out_specs=[pl.BlockSpec((blk, HF), lambda i: (i, 0))] * 4,
        out_shape=[half] * 4,
    )(x_pad, Wq, Wk)


def _pass1_body(q0_hbm, q1_hbm, k0_hbm, k1_hbm, row_hbm, col_hbm, w_hbm,
                ridx, cidx, qa, qb, ka, kb, wv, sem):
    wid = lax.axis_index("s") * NC + lax.axis_index("c")
    base = wid * (P1_CHUNKS * CH)

    @pl.loop(0, P1_CHUNKS)
    def _chunk(i):
        off = base + i * CH
        pltpu.sync_copy(row_hbm.at[pl.ds(off, CH)], ridx)
        pltpu.sync_copy(col_hbm.at[pl.ds(off, CH)], cidx)
        c1 = pltpu.async_copy(q0_hbm.at[ridx], qa, sem)
        c2 = pltpu.async_copy(q1_hbm.at[ridx], qb, sem)
        c3 = pltpu.async_copy(k0_hbm.at[cidx], ka, sem)
        c4 = pltpu.async_copy(k1_hbm.at[cidx], kb, sem)
        c1.wait()
        c2.wait()
        c3.wait()
        c4.wait()

        lane = lax.iota(jnp.int32, 16)

        @pl.loop(0, CH // 16)
        def _grp(g):
            dots = jnp.zeros((16,), jnp.float32)
            for j in range(16):
                e = g * 16 + j
                acc = qa[e, pl.ds(0, 16)] * ka[e, pl.ds(0, 16)]
                for t in range(1, 8):
                    acc = acc + qa[e, pl.ds(t * 16, 16)] * ka[e, pl.ds(t * 16, 16)]
                for t in range(8):
                    acc = acc + qb[e, pl.ds(t * 16, 16)] * kb[e, pl.ds(t * 16, 16)]
                dots = jnp.where(lane == j, jnp.sum(acc), dots)
            wv[pl.ds(g * 16, 16)] = jnp.exp(dots * (1.0 / SCALE))

        pltpu.sync_copy(wv, w_hbm.at[pl.ds(off, CH)])


def _pass1(q0, q1, k0, k1, row_p, col_p):
    kfn = pl.kernel(
        _pass1_body,
        out_type=jax.ShapeDtypeStruct((EPAD,), jnp.float32),
        mesh=_mesh,
        compiler_params=_sc_params,
        scratch_types=[
            pltpu.VMEM((CH,), jnp.int32),
            pltpu.VMEM((CH,), jnp.int32),
            pltpu.VMEM((CH, HF), jnp.float32),
            pltpu.VMEM((CH, HF), jnp.float32),
            pltpu.VMEM((CH, HF), jnp.float32),
            pltpu.VMEM((CH, HF), jnp.float32),
            pltpu.VMEM((CH,), jnp.float32),
            pltpu.SemaphoreType.DMA,
        ],
    )
    return kfn(q0, q1, k0, k1, row_p, col_p)


def _pass2_body(q0_hbm, q1_hbm, row_hbm, col_hbm, w_hbm,
                num0_hbm, num1_hbm, den_hbm,
                ridx, cidx, wv, qv, zv, zd, snum, sden, sem):
    cid = lax.axis_index("c")
    sid = lax.axis_index("s")

    @pl.loop(0, CH)
    def _z(r):
        for t in range(HF // 16):
            zv[r, pl.ds(t * 16, 16)] = jnp.zeros((16,), jnp.float32)

    @pl.loop(0, ROWS_PER_TILE // 16)
    def _zd(i):
        zd[pl.ds(i * 16, 16)] = jnp.zeros((16,), jnp.float32)

    @pl.loop(0, ROWS_PER_TILE // CH)
    def _zs(b):
        pltpu.sync_copy(zv, snum.at[pl.ds(sid * ROWS_PER_TILE + b * CH, CH)])

    pltpu.sync_copy(zd, sden.at[pl.ds(sid * ROWS_PER_TILE, ROWS_PER_TILE)])
    plsc.subcore_barrier()

    base = sid * (P2_CHUNKS * CH)

    def run(q_hbm):
        @pl.loop(0, P2_CHUNKS)
        def _chunk(i):
            off = base + i * CH
            pltpu.sync_copy(row_hbm.at[pl.ds(off, CH)], ridx)
            pltpu.sync_copy(col_hbm.at[pl.ds(off, CH)], cidx)
            pltpu.sync_copy(w_hbm.at[pl.ds(off, CH)], wv)
            pltpu.async_copy(q_hbm.at[ridx], qv, sem).wait()

            @pl.loop(0, CH // 16)
            def _grp(g):
                ws = wv[pl.ds(g * 16, 16)]
                for j in range(16):
                    e = g * 16 + j
                    we = ws[j]
                    for t in range(HF // 16):
                        qv[e, pl.ds(t * 16, 16)] = qv[e, pl.ds(t * 16, 16)] * we

            pltpu.sync_copy(qv, snum.at[cidx], add=True)
            pltpu.sync_copy(wv, sden.at[cidx], add=True)

    @pl.when(cid == 0)
    def _c0():
        run(q0_hbm)

    @pl.when(cid == 1)
    def _c1():
        run(q1_hbm)

    plsc.subcore_barrier()

    @pl.loop(0, ROWS_PER_TILE // CH)
    def _wb(b):
        r0 = sid * ROWS_PER_TILE + b * CH

        @pl.when(cid == 0)
        def _w0():
            pltpu.sync_copy(snum.at[pl.ds(r0, CH)], num0_hbm.at[pl.ds(r0, CH)])

        @pl.when(cid == 1)
        def _w1():
            pltpu.sync_copy(snum.at[pl.ds(r0, CH)], num1_hbm.at[pl.ds(r0, CH)])

    @pl.when(cid == 0)
    def _wd():
        pltpu.sync_copy(sden.at[pl.ds(sid * ROWS_PER_TILE, ROWS_PER_TILE)],
                        den_hbm.at[pl.ds(sid * ROWS_PER_TILE, ROWS_PER_TILE)])


def _pass2(q0, q1, row_p, col_p, w):
    half = jax.ShapeDtypeStruct((NPAD, HF), jnp.float32)
    kfn = pl.kernel(
        _pass2_body,
        out_type=(half, half, jax.ShapeDtypeStruct((NPAD,), jnp.float32)),
        mesh=_mesh,
        compiler_params=_sc_params,
        scratch_types=[
            pltpu.VMEM((CH,), jnp.int32),
            pltpu.VMEM((CH,), jnp.int32),
            pltpu.VMEM((CH,), jnp.float32),
            pltpu.VMEM((CH, HF), jnp.float32),
            pltpu.VMEM((CH, HF), jnp.float32),
            pltpu.VMEM((ROWS_PER_TILE,), jnp.float32),
            pltpu.VMEM_SHARED((NPAD, HF), jnp.float32),
            pltpu.VMEM_SHARED((NPAD,), jnp.float32),
            pltpu.SemaphoreType.DMA,
        ],
    )
    return kfn(q0, q1, row_p, col_p, w)


def _div_body(n0_ref, n1_ref, d_ref, o_ref):
    d = d_ref[...] + 1e-16
    o_ref[:, :HF] = n0_ref[...] / d
    o_ref[:, HF:] = n1_ref[...] / d


def _divide(num0, num1, den2d):
    blk = 1024
    return pl.pallas_call(
        _div_body,
        grid=(NPAD // blk,),
        in_specs=[
            pl.BlockSpec((blk, HF), lambda i: (i, 0)),
            pl.BlockSpec((blk, HF), lambda i: (i, 0)),
            pl.BlockSpec((blk, 1), lambda i: (i, 0)),
        ],
        out_specs=pl.BlockSpec((blk, F), lambda i: (i, 0)),
        out_shape=jax.ShapeDtypeStruct((NPAD, F), jnp.float32),
    )(num0, num1, den2d)


def kernel(x, Wq, Wk, edge_index):
    row = edge_index[0].astype(jnp.int32)
    col = edge_index[1].astype(jnp.int32)
    pad = jnp.full((EPAD - E,), N, dtype=jnp.int32)
    row_p = jnp.concatenate([row, pad])
    col_p = jnp.concatenate([col, pad])
    x_pad = jnp.pad(x, ((0, NPAD - N), (0, 0)))

    q0, q1, k0, k1 = _proj(x_pad, Wq, Wk)
    w = _pass1(q0, q1, k0, k1, row_p, col_p)
    num0, num1, den = _pass2(q0, q1, row_p, col_p, w)
    out = _divide(num0, num1, den.reshape(NPAD, 1))
    return out[:N]


# speedup vs baseline: 4.0026x; 1.6327x over previous
import dataclasses
import functools
import math

import jax
import jax.numpy as jnp
from jax import lax
from jax.experimental import pallas as pl
from jax.experimental.pallas import tpu as pltpu
from jax.experimental.pallas import tpu_sc as plsc

N = 10000
E = 160000
F = 256
HF = 128
SCALE = math.sqrt(F)

NPAD = 10240
EPAD = 163840
NC = 2
NS = 16

C1 = 64
P1_CHUNKS = EPAD // (NC * NS) // C1
C2 = 128
P2_CHUNKS = EPAD // NS // C2
ROWS_PER_TILE = NPAD // NS

_mesh = plsc.VectorSubcoreMesh(core_axis_name="c", subcore_axis_name="s")

_sc_params = pltpu.CompilerParams()
if "needs_layout_passes" in pltpu.CompilerParams.__dataclass_fields__:
    _sc_params = dataclasses.replace(_sc_params, needs_layout_passes=False)


def _proj_body(x_ref, wq_ref, wk_ref, q_ref, k_ref, q0_ref, q1_ref):
    xb = x_ref[...]
    dn = (((1,), (1,)), ((), ()))
    q = lax.dot_general(xb, wq_ref[...], dn,
                        preferred_element_type=jnp.float32,
                        precision=lax.Precision.HIGHEST)
    k = lax.dot_general(xb, wk_ref[...], dn,
                        preferred_element_type=jnp.float32,
                        precision=lax.Precision.HIGHEST)
    q_ref[...] = q
    k_ref[...] = k
    q0_ref[...] = q[:, :HF]
    q1_ref[...] = q[:, HF:]


def _proj(x_pad, Wq, Wk):
    blk = 1024
    full = jax.ShapeDtypeStruct((NPAD, F), jnp.float32)
    half = jax.ShapeDtypeStruct((NPAD, HF), jnp.float32)
    return pl.pallas_call(
        _proj_body,
        grid=(NPAD // blk,),
        in_specs=[
            pl.BlockSpec((blk, F), lambda i: (i, 0)),
            pl.BlockSpec((F, F), lambda i: (0, 0)),
            pl.BlockSpec((F, F), lambda i: (0, 0)),
        ],
        out_specs=[
            pl.BlockSpec((blk, F), lambda i: (i, 0)),
            pl.BlockSpec((blk, F), lambda i: (i, 0)),
            pl.BlockSpec((blk, HF), lambda i: (i, 0)),
            pl.BlockSpec((blk, HF), lambda i: (i, 0)),
        ],
        out_shape=[full, full, half, half],
    )(x_pad, Wq, Wk)


def _p1_start(q_hbm, k_hbm, row_hbm, col_hbm, off, ridx, cidx, qv, kv, sem):
    pltpu.sync_copy(row_hbm.at[pl.ds(off, C1)], ridx)
    pltpu.sync_copy(col_hbm.at[pl.ds(off, C1)], cidx)
    pltpu.async_copy(q_hbm.at[ridx], qv, sem)
    pltpu.async_copy(k_hbm.at[cidx], kv, sem)


def _p1_wait(q_hbm, k_hbm, ridx, cidx, qv, kv, sem):
    pltpu.make_async_copy(q_hbm.at[ridx], qv, sem).wait()
    pltpu.make_async_copy(k_hbm.at[cidx], kv, sem).wait()


def _p1_compute(qv, kv, wv, w_hbm, off):
    lane = lax.iota(jnp.int32, 16)

    @pl.loop(0, C1 // 16)
    def _grp(g):
        dots = jnp.zeros((16,), jnp.float32)
        for j in range(16):
            e = g * 16 + j
            acc = qv[e, pl.ds(0, 16)] * kv[e, pl.ds(0, 16)]
            for t in range(1, F // 16):
                acc = acc + qv[e, pl.ds(t * 16, 16)] * kv[e, pl.ds(t * 16, 16)]
            dots = jnp.where(lane == j, jnp.sum(acc), dots)
        wv[pl.ds(g * 16, 16)] = jnp.exp(dots * (1.0 / SCALE))

    pltpu.sync_copy(wv, w_hbm.at[pl.ds(off, C1)])


def _pass1_body(q_hbm, k_hbm, row_hbm, col_hbm, w_hbm,
                ridx0, cidx0, ridx1, cidx1, qv0, kv0, qv1, kv1, wv,
                sem0, sem1):
    wid = lax.axis_index("s") * NC + lax.axis_index("c")
    base = wid * (P1_CHUNKS * C1)
    npairs = P1_CHUNKS // 2

    _p1_start(q_hbm, k_hbm, row_hbm, col_hbm, base, ridx0, cidx0, qv0, kv0, sem0)

    @pl.loop(0, npairs)
    def _pair(i):
        off0 = base + (2 * i) * C1
        _p1_start(q_hbm, k_hbm, row_hbm, col_hbm, off0 + C1,
                  ridx1, cidx1, qv1, kv1, sem1)
        _p1_wait(q_hbm, k_hbm, ridx0, cidx0, qv0, kv0, sem0)
        _p1_compute(qv0, kv0, wv, w_hbm, off0)

        @pl.when(i < npairs - 1)
        def _pref():
            _p1_start(q_hbm, k_hbm, row_hbm, col_hbm, off0 + 2 * C1,
                      ridx0, cidx0, qv0, kv0, sem0)

        _p1_wait(q_hbm, k_hbm, ridx1, cidx1, qv1, kv1, sem1)
        _p1_compute(qv1, kv1, wv, w_hbm, off0 + C1)


def _pass1(q, k, row_p, col_p):
    kfn = pl.kernel(
        _pass1_body,
        out_type=jax.ShapeDtypeStruct((EPAD,), jnp.float32),
        mesh=_mesh,
        compiler_params=_sc_params,
        scratch_types=[
            pltpu.VMEM((C1,), jnp.int32),
            pltpu.VMEM((C1,), jnp.int32),
            pltpu.VMEM((C1,), jnp.int32),
            pltpu.VMEM((C1,), jnp.int32),
            pltpu.VMEM((C1, F), jnp.float32),
            pltpu.VMEM((C1, F), jnp.float32),
            pltpu.VMEM((C1, F), jnp.float32),
            pltpu.VMEM((C1, F), jnp.float32),
            pltpu.VMEM((C1,), jnp.float32),
            pltpu.SemaphoreType.DMA,
            pltpu.SemaphoreType.DMA,
        ],
    )
    return kfn(q, k, row_p, col_p)


def _p2_start(q_hbm, row_hbm, col_hbm, w_hbm, off, ridx, cidx, wv, qv, sem):
    pltpu.sync_copy(row_hbm.at[pl.ds(off, C2)], ridx)
    pltpu.sync_copy(col_hbm.at[pl.ds(off, C2)], cidx)
    pltpu.sync_copy(w_hbm.at[pl.ds(off, C2)], wv)
    pltpu.async_copy(q_hbm.at[ridx], qv, sem)


def _p2_scale(qv, wv):
    @pl.loop(0, C2 // 16)
    def _grp(g):
        ws = wv[pl.ds(g * 16, 16)]
        for j in range(16):
            e = g * 16 + j
            we = ws[j]
            for t in range(HF // 16):
                qv[e, pl.ds(t * 16, 16)] = qv[e, pl.ds(t * 16, 16)] * we


def _p2_scatter(qv, wv, cidx, snum, sden):
    pltpu.sync_copy(qv, snum.at[cidx], add=True)
    pltpu.sync_copy(wv, sden.at[cidx], add=True)


def _pass2_body(q0_hbm, q1_hbm, row_hbm, col_hbm, w_hbm,
                num0_hbm, num1_hbm, den_hbm,
                ridx0, cidx0, wv0, qv0, ridx1, cidx1, wv1, qv1,
                zd, snum, sden, gsem0, gsem1):
    cid = lax.axis_index("c")
    sid = lax.axis_index("s")

    @pl.loop(0, C2)
    def _z(r):
        for t in range(HF // 16):
            qv0[r, pl.ds(t * 16, 16)] = jnp.zeros((16,), jnp.float32)

    @pl.loop(0, ROWS_PER_TILE // 16)
    def _zd(i):
        zd[pl.ds(i * 16, 16)] = jnp.zeros((16,), jnp.float32)

    @pl.loop(0, ROWS_PER_TILE // C2)
    def _zs(b):
        pltpu.sync_copy(qv0, snum.at[pl.ds(sid * ROWS_PER_TILE + b * C2, C2)])

    pltpu.sync_copy(zd, sden.at[pl.ds(sid * ROWS_PER_TILE, ROWS_PER_TILE)])
    plsc.subcore_barrier()

    base = sid * (P2_CHUNKS * C2)
    npairs = P2_CHUNKS // 2

    def run(q_hbm):
        _p2_start(q_hbm, row_hbm, col_hbm, w_hbm, base,
                  ridx0, cidx0, wv0, qv0, gsem0)

        @pl.loop(0, npairs)
        def _pair(i):
            off0 = base + (2 * i) * C2
            _p2_start(q_hbm, row_hbm, col_hbm, w_hbm, off0 + C2,
                      ridx1, cidx1, wv1, qv1, gsem1)
            pltpu.make_async_copy(q_hbm.at[ridx0], qv0, gsem0).wait()
            _p2_scale(qv0, wv0)
            _p2_scatter(qv0, wv0, cidx0, snum, sden)

            @pl.when(i < npairs - 1)
            def _pref():
                _p2_start(q_hbm, row_hbm, col_hbm, w_hbm, off0 + 2 * C2,
                          ridx0, cidx0, wv0, qv0, gsem0)

            pltpu.make_async_copy(q_hbm.at[ridx1], qv1, gsem1).wait()
            _p2_scale(qv1, wv1)
            _p2_scatter(qv1, wv1, cidx1, snum, sden)

    @pl.when(cid == 0)
    def _c0():
        run(q0_hbm)

    @pl.when(cid == 1)
    def _c1():
        run(q1_hbm)

    plsc.subcore_barrier()

    @pl.loop(0, ROWS_PER_TILE // C2)
    def _wb(b):
        r0 = sid * ROWS_PER_TILE + b * C2

        @pl.when(cid == 0)
        def _w0():
            pltpu.sync_copy(snum.at[pl.ds(r0, C2)], num0_hbm.at[pl.ds(r0, C2)])

        @pl.when(cid == 1)
        def _w1():
            pltpu.sync_copy(snum.at[pl.ds(r0, C2)], num1_hbm.at[pl.ds(r0, C2)])

    @pl.when(cid == 0)
    def _wd():
        pltpu.sync_copy(sden.at[pl.ds(sid * ROWS_PER_TILE, ROWS_PER_TILE)],
                        den_hbm.at[pl.ds(sid * ROWS_PER_TILE, ROWS_PER_TILE)])


def _pass2(q0, q1, row_p, col_p, w):
    half = jax.ShapeDtypeStruct((NPAD, HF), jnp.float32)
    kfn = pl.kernel(
        _pass2_body,
        out_type=(half, half, jax.ShapeDtypeStruct((NPAD,), jnp.float32)),
        mesh=_mesh,
        compiler_params=_sc_params,
        scratch_types=[
            pltpu.VMEM((C2,), jnp.int32),
            pltpu.VMEM((C2,), jnp.int32),
            pltpu.VMEM((C2,), jnp.float32),
            pltpu.VMEM((C2, HF), jnp.float32),
            pltpu.VMEM((C2,), jnp.int32),
            pltpu.VMEM((C2,), jnp.int32),
            pltpu.VMEM((C2,), jnp.float32),
            pltpu.VMEM((C2, HF), jnp.float32),
            pltpu.VMEM((ROWS_PER_TILE,), jnp.float32),
            pltpu.VMEM_SHARED((NPAD, HF), jnp.float32),
            pltpu.VMEM_SHARED((NPAD,), jnp.float32),
            pltpu.SemaphoreType.DMA,
            pltpu.SemaphoreType.DMA,
        ],
    )
    return kfn(q0, q1, row_p, col_p, w)


def _div_body(n0_ref, n1_ref, d_ref, o_ref):
    d = d_ref[...] + 1e-16
    o_ref[:, :HF] = n0_ref[...] / d
    o_ref[:, HF:] = n1_ref[...] / d


def _divide(num0, num1, den2d):
    blk = 1024
    return pl.pallas_call(
        _div_body,
        grid=(NPAD // blk,),
        in_specs=[
            pl.BlockSpec((blk, HF), lambda i: (i, 0)),
            pl.BlockSpec((blk, HF), lambda i: (i, 0)),
            pl.BlockSpec((blk, 1), lambda i: (i, 0)),
        ],
        out_specs=pl.BlockSpec((blk, F), lambda i: (i, 0)),
        out_shape=jax.ShapeDtypeStruct((NPAD, F), jnp.float32),
    )(num0, num1, den2d)


def kernel(x, Wq, Wk, edge_index):
    row = edge_index[0].astype(jnp.int32)
    col = edge_index[1].astype(jnp.int32)
    pad = jnp.full((EPAD - E,), N, dtype=jnp.int32)
    row_p = jnp.concatenate([row, pad])
    col_p = jnp.concatenate([col, pad])
    x_pad = jnp.pad(x, ((0, NPAD - N), (0, 0)))

    q, k, q0, q1 = _proj(x_pad, Wq, Wk)
    w = _pass1(q, k, row_p, col_p)
    num0, num1, den = _pass2(q0, q1, row_p, col_p, w)
    out = _divide(num0, num1, den.reshape(NPAD, 1))
    return out[:N]


# speedup vs baseline: 4.5852x; 1.1455x over previous
import dataclasses
import functools
import math

import jax
import jax.numpy as jnp
from jax import lax
from jax.experimental import pallas as pl
from jax.experimental.pallas import tpu as pltpu
from jax.experimental.pallas import tpu_sc as plsc

N = 10000
E = 160000
F = 256
HF = 128
SCALE = math.sqrt(F)

NPAD = 10240
EPAD = 163840
NC = 2
NS = 16
PACK_SHIFT = 14

E_TILE1 = EPAD // (NC * NS)
C1 = 80
P1_CHUNKS = E_TILE1 // C1

E_TILE2 = EPAD // NS
C2 = 64
P2_CHUNKS = E_TILE2 // C2
ROWS_PER_TILE = NPAD // NS

_mesh = plsc.VectorSubcoreMesh(core_axis_name="c", subcore_axis_name="s")

_sc_params = pltpu.CompilerParams()
if "needs_layout_passes" in pltpu.CompilerParams.__dataclass_fields__:
    _sc_params = dataclasses.replace(_sc_params, needs_layout_passes=False)


def _unpack_idx(packed_all, off, ridx, cidx, n):
    @pl.loop(0, n // 16)
    def _grp(g):
        p = packed_all[pl.ds(off + g * 16, 16)]
        ridx[pl.ds(g * 16, 16)] = lax.shift_right_logical(p, PACK_SHIFT)
        cidx[pl.ds(g * 16, 16)] = lax.bitwise_and(p, (1 << PACK_SHIFT) - 1)


def _proj_body(x_ref, wq_ref, wk_ref, q_ref, k_ref, q0_ref, q1_ref):
    xb = x_ref[...]
    dn = (((1,), (1,)), ((), ()))
    q = lax.dot_general(xb, wq_ref[...], dn,
                        preferred_element_type=jnp.float32,
                        precision=lax.Precision.HIGHEST)
    k = lax.dot_general(xb, wk_ref[...], dn,
                        preferred_element_type=jnp.float32,
                        precision=lax.Precision.HIGHEST)
    q_ref[...] = q
    k_ref[...] = k
    q0_ref[...] = q[:, :HF]
    q1_ref[...] = q[:, HF:]


def _proj(x_pad, Wq, Wk):
    blk = 1024
    full = jax.ShapeDtypeStruct((NPAD, F), jnp.float32)
    half = jax.ShapeDtypeStruct((NPAD, HF), jnp.float32)
    return pl.pallas_call(
        _proj_body,
        grid=(NPAD // blk,),
        in_specs=[
            pl.BlockSpec((blk, F), lambda i: (i, 0)),
            pl.BlockSpec((F, F), lambda i: (0, 0)),
            pl.BlockSpec((F, F), lambda i: (0, 0)),
        ],
        out_specs=[
            pl.BlockSpec((blk, F), lambda i: (i, 0)),
            pl.BlockSpec((blk, F), lambda i: (i, 0)),
            pl.BlockSpec((blk, HF), lambda i: (i, 0)),
            pl.BlockSpec((blk, HF), lambda i: (i, 0)),
        ],
        out_shape=[full, full, half, half],
    )(x_pad, Wq, Wk)


def _p1_start(q_hbm, k_hbm, packed, off, ridx, cidx, qv, kv, sem):
    _unpack_idx(packed, off, ridx, cidx, C1)
    pltpu.async_copy(q_hbm.at[ridx], qv, sem)
    pltpu.async_copy(k_hbm.at[cidx], kv, sem)


def _p1_wait(q_hbm, k_hbm, ridx, cidx, qv, kv, sem):
    pltpu.make_async_copy(q_hbm.at[ridx], qv, sem).wait()
    pltpu.make_async_copy(k_hbm.at[cidx], kv, sem).wait()


def _p1_compute(qv, kv, wv_all, off):
    lane = lax.iota(jnp.int32, 16)

    @pl.loop(0, C1 // 16)
    def _grp(g):
        dots = jnp.zeros((16,), jnp.float32)
        for j in range(16):
            e = g * 16 + j
            acc = qv[e, pl.ds(0, 16)] * kv[e, pl.ds(0, 16)]
            for t in range(1, F // 16):
                acc = acc + qv[e, pl.ds(t * 16, 16)] * kv[e, pl.ds(t * 16, 16)]
            dots = jnp.where(lane == j, jnp.sum(acc), dots)
        wv_all[pl.ds(off + g * 16, 16)] = jnp.exp(dots * (1.0 / SCALE))


def _pass1_body(q_hbm, k_hbm, packed_hbm, w_hbm,
                packed, wv_all, ridx0, cidx0, ridx1, cidx1,
                qv0, kv0, qv1, kv1, sem0, sem1):
    wid = lax.axis_index("s") * NC + lax.axis_index("c")
    base = wid * E_TILE1
    npairs = P1_CHUNKS // 2

    pltpu.sync_copy(packed_hbm.at[pl.ds(base, E_TILE1)], packed)
    _p1_start(q_hbm, k_hbm, packed, 0, ridx0, cidx0, qv0, kv0, sem0)

    @pl.loop(0, npairs)
    def _pair(i):
        off0 = (2 * i) * C1
        _p1_start(q_hbm, k_hbm, packed, off0 + C1, ridx1, cidx1, qv1, kv1, sem1)
        _p1_wait(q_hbm, k_hbm, ridx0, cidx0, qv0, kv0, sem0)
        _p1_compute(qv0, kv0, wv_all, off0)

        @pl.when(i < npairs - 1)
        def _pref():
            _p1_start(q_hbm, k_hbm, packed, off0 + 2 * C1,
                      ridx0, cidx0, qv0, kv0, sem0)

        _p1_wait(q_hbm, k_hbm, ridx1, cidx1, qv1, kv1, sem1)
        _p1_compute(qv1, kv1, wv_all, off0 + C1)

    pltpu.sync_copy(wv_all, w_hbm.at[pl.ds(base, E_TILE1)])


def _pass1(q, k, packed_p):
    kfn = pl.kernel(
        _pass1_body,
        out_type=jax.ShapeDtypeStruct((EPAD,), jnp.float32),
        mesh=_mesh,
        compiler_params=_sc_params,
        scratch_types=[
            pltpu.VMEM((E_TILE1,), jnp.int32),
            pltpu.VMEM((E_TILE1,), jnp.float32),
            pltpu.VMEM((C1,), jnp.int32),
            pltpu.VMEM((C1,), jnp.int32),
            pltpu.VMEM((C1,), jnp.int32),
            pltpu.VMEM((C1,), jnp.int32),
            pltpu.VMEM((C1, F), jnp.float32),
            pltpu.VMEM((C1, F), jnp.float32),
            pltpu.VMEM((C1, F), jnp.float32),
            pltpu.VMEM((C1, F), jnp.float32),
            pltpu.SemaphoreType.DMA,
            pltpu.SemaphoreType.DMA,
        ],
    )
    return kfn(q, k, packed_p)


def _p2_scale(qv, wv_all, off):
    @pl.loop(0, C2 // 16)
    def _grp(g):
        ws = wv_all[pl.ds(off + g * 16, 16)]
        for j in range(16):
            e = g * 16 + j
            we = ws[j]
            for t in range(HF // 16):
                qv[e, pl.ds(t * 16, 16)] = qv[e, pl.ds(t * 16, 16)] * we


def _p2_scatter_start(qv, wv_all, off, cidx, snum, sden, sem):
    pltpu.async_copy(qv, snum.at[cidx], sem, add=True)
    pltpu.async_copy(wv_all.at[pl.ds(off, C2)], sden.at[cidx], sem, add=True)


def _p2_scatter_wait(qv, wv_all, off, cidx, snum, sden, sem):
    pltpu.make_async_copy(qv, snum.at[cidx], sem).wait()
    pltpu.make_async_copy(wv_all.at[pl.ds(off, C2)], sden.at[cidx], sem).wait()


def _pass2_body(q0_hbm, q1_hbm, packed_hbm, w_hbm,
                num0_hbm, num1_hbm, den_hbm,
                packed, wv_all, ridx0, cidx0, ridx1, cidx1, qv0, qv1, zd,
                snum, sden, gsem0, gsem1, ssem0, ssem1):
    cid = lax.axis_index("c")
    sid = lax.axis_index("s")
    base = sid * E_TILE2

    pltpu.sync_copy(packed_hbm.at[pl.ds(base, E_TILE2)], packed)
    pltpu.sync_copy(w_hbm.at[pl.ds(base, E_TILE2)], wv_all)

    @pl.loop(0, C2)
    def _z(r):
        for t in range(HF // 16):
            qv0[r, pl.ds(t * 16, 16)] = jnp.zeros((16,), jnp.float32)

    @pl.loop(0, ROWS_PER_TILE // 16)
    def _zd(i):
        zd[pl.ds(i * 16, 16)] = jnp.zeros((16,), jnp.float32)

    @pl.loop(0, ROWS_PER_TILE // C2)
    def _zs(b):
        pltpu.sync_copy(qv0, snum.at[pl.ds(sid * ROWS_PER_TILE + b * C2, C2)])

    pltpu.sync_copy(zd, sden.at[pl.ds(sid * ROWS_PER_TILE, ROWS_PER_TILE)])
    plsc.subcore_barrier()

    npairs = P2_CHUNKS // 2

    def run(q_hbm):
        _unpack_idx(packed, 0, ridx0, cidx0, C2)
        pltpu.async_copy(q_hbm.at[ridx0], qv0, gsem0)

        @pl.loop(0, npairs)
        def _pair(i):
            off0 = (2 * i) * C2

            @pl.when(i > 0)
            def _ws1():
                _p2_scatter_wait(qv1, wv_all, off0 - C2, cidx1, snum, sden, ssem1)

            _unpack_idx(packed, off0 + C2, ridx1, cidx1, C2)
            pltpu.async_copy(q_hbm.at[ridx1], qv1, gsem1)

            pltpu.make_async_copy(q_hbm.at[ridx0], qv0, gsem0).wait()
            _p2_scale(qv0, wv_all, off0)
            _p2_scatter_start(qv0, wv_all, off0, cidx0, snum, sden, ssem0)

            @pl.when(i < npairs - 1)
            def _pref():
                _p2_scatter_wait(qv0, wv_all, off0, cidx0, snum, sden, ssem0)
                _unpack_idx(packed, off0 + 2 * C2, ridx0, cidx0, C2)
                pltpu.async_copy(q_hbm.at[ridx0], qv0, gsem0)

            pltpu.make_async_copy(q_hbm.at[ridx1], qv1, gsem1).wait()
            _p2_scale(qv1, wv_all, off0 + C2)
            _p2_scatter_start(qv1, wv_all, off0 + C2, cidx1, snum, sden, ssem1)

        _p2_scatter_wait(qv0, wv_all, (P2_CHUNKS - 2) * C2, cidx0,
                         snum, sden, ssem0)
        _p2_scatter_wait(qv1, wv_all, (P2_CHUNKS - 1) * C2, cidx1,
                         snum, sden, ssem1)

    @pl.when(cid == 0)
    def _c0():
        run(q0_hbm)

    @pl.when(cid == 1)
    def _c1():
        run(q1_hbm)

    plsc.subcore_barrier()

    @pl.loop(0, ROWS_PER_TILE // C2)
    def _wb(b):
        r0 = sid * ROWS_PER_TILE + b * C2

        @pl.when(cid == 0)
        def _w0():
            pltpu.sync_copy(snum.at[pl.ds(r0, C2)], num0_hbm.at[pl.ds(r0, C2)])

        @pl.when(cid == 1)
        def _w1():
            pltpu.sync_copy(snum.at[pl.ds(r0, C2)], num1_hbm.at[pl.ds(r0, C2)])

    @pl.when(cid == 0)
    def _wd():
        pltpu.sync_copy(sden.at[pl.ds(sid * ROWS_PER_TILE, ROWS_PER_TILE)],
                        den_hbm.at[pl.ds(sid * ROWS_PER_TILE, ROWS_PER_TILE)])


def _pass2(q0, q1, packed_p, w):
    half = jax.ShapeDtypeStruct((NPAD, HF), jnp.float32)
    kfn = pl.kernel(
        _pass2_body,
        out_type=(half, half, jax.ShapeDtypeStruct((NPAD,), jnp.float32)),
        mesh=_mesh,
        compiler_params=_sc_params,
        scratch_types=[
            pltpu.VMEM((E_TILE2,), jnp.int32),
            pltpu.VMEM((E_TILE2,), jnp.float32),
            pltpu.VMEM((C2,), jnp.int32),
            pltpu.VMEM((C2,), jnp.int32),
            pltpu.VMEM((C2,), jnp.int32),
            pltpu.VMEM((C2,), jnp.int32),
            pltpu.VMEM((C2, HF), jnp.float32),
            pltpu.VMEM((C2, HF), jnp.float32),
            pltpu.VMEM((ROWS_PER_TILE,), jnp.float32),
            pltpu.VMEM_SHARED((NPAD, HF), jnp.float32),
            pltpu.VMEM_SHARED((NPAD,), jnp.float32),
            pltpu.SemaphoreType.DMA,
            pltpu.SemaphoreType.DMA,
            pltpu.SemaphoreType.DMA,
            pltpu.SemaphoreType.DMA,
        ],
    )
    return kfn(q0, q1, packed_p, w)


def _div_body(n0_ref, n1_ref, d_ref, o_ref):
    d = d_ref[...] + 1e-16
    o_ref[:, :HF] = n0_ref[...] / d
    o_ref[:, HF:] = n1_ref[...] / d


def _divide(num0, num1, den2d):
    blk = 1024
    return pl.pallas_call(
        _div_body,
        grid=(NPAD // blk,),
        in_specs=[
            pl.BlockSpec((blk, HF), lambda i: (i, 0)),
            pl.BlockSpec((blk, HF), lambda i: (i, 0)),
            pl.BlockSpec((blk, 1), lambda i: (i, 0)),
        ],
        out_specs=pl.BlockSpec((blk, F), lambda i: (i, 0)),
        out_shape=jax.ShapeDtypeStruct((NPAD, F), jnp.float32),
    )(num0, num1, den2d)


def kernel(x, Wq, Wk, edge_index):
    row = edge_index[0].astype(jnp.int32)
    col = edge_index[1].astype(jnp.int32)
    pad = jnp.full((EPAD - E,), N, dtype=jnp.int32)
    row_p = jnp.concatenate([row, pad])
    col_p = jnp.concatenate([col, pad])
    packed_p = (row_p << PACK_SHIFT) | col_p
    x_pad = jnp.pad(x, ((0, NPAD - N), (0, 0)))

    q, k, q0, q1 = _proj(x_pad, Wq, Wk)
    w = _pass1(q, k, packed_p)
    num0, num1, den = _pass2(q0, q1, packed_p, w)
    out = _divide(num0, num1, den.reshape(NPAD, 1))
    return out[:N]
